# Initial kernel scaffold; baseline (speedup 1.0000x reference)
import jax
import jax.numpy as jnp
from jax.experimental import pallas as pl


def kernel(x, Wr, Wg, Wu, Wd, Wsg, Wsu, Wsd):
    raise NotImplementedError("write your pallas kernel here")


# speedup vs baseline: 1.1749x; 1.1749x over previous
import functools

import jax
import jax.numpy as jnp
from jax import lax
from jax.experimental import pallas as pl
from jax.experimental.pallas import tpu as pltpu
from jax.experimental.pallas import tpu_sc as plsc

D = 768
F = 768
E = 8
S = 2048
T = 128
SPAD = S + E * T
NB = SPAD // T
NC = 2
NS = 16
NW = NC * NS
RPW = SPAD // NW
CHUNK = 512


def _router_kernel(x_ref, wr_ref, src_ref, dst_ref, bexp_ref):
    x = x_ref[...]
    logits = jnp.dot(x, wr_ref[...],
                     preferred_element_type=jnp.float32)
    m = jnp.max(logits, axis=1, keepdims=True)
    ex = jnp.exp(logits - m)
    probs = ex / jnp.sum(ex, axis=1, keepdims=True)
    pm = jnp.max(probs, axis=1, keepdims=True)
    eidx = lax.broadcasted_iota(jnp.int32, (S, E), 1)
    ids2 = jnp.where(probs == pm, eidx, E)
    eid = jnp.min(ids2, axis=1)
    oh = (eid[:, None] == eidx).astype(jnp.float32)

    counts = jnp.round(jnp.sum(oh, axis=0)).astype(jnp.int32)
    padded = ((counts + T - 1) // T) * T
    lt = (lax.broadcasted_iota(jnp.int32, (E, E), 1)
          < lax.broadcasted_iota(jnp.int32, (E, E), 0))
    starts = jnp.sum(jnp.where(lt, padded[None, :], 0), axis=1)
    cum_incl = starts + padded

    r128 = lax.broadcasted_iota(jnp.int32, (T, T), 0)
    c128 = lax.broadcasted_iota(jnp.int32, (T, T), 1)
    tri = (c128 < r128).astype(jnp.float32)
    base = jnp.zeros((1, E), jnp.float32)
    pos_parts = []
    startsf = starts.astype(jnp.float32)
    for k in range(S // T):
        ohk = oh[k * T:(k + 1) * T]
        rk = jnp.dot(tri, ohk, preferred_element_type=jnp.float32)
        full = rk + base + startsf[None, :]
        pos_parts.append(jnp.sum(full * ohk, axis=1))
        base = base + jnp.sum(ohk, axis=0, keepdims=True)
    pos = jnp.concatenate(pos_parts, axis=0)

    ti = lax.broadcasted_iota(jnp.int32, (1, S), 1)
    tq = (ti // 16).astype(jnp.float32)
    tr = (ti % 16).astype(jnp.float32)
    ones = jnp.ones((1, S), jnp.float32)
    lhs = jnp.concatenate([tq, tr, ones], axis=0)
    src_parts, dst_parts = [], []
    for c in range(SPAD // CHUNK):
        cols = (c * CHUNK
                + lax.broadcasted_iota(jnp.int32, (1, CHUNK), 1)
                ).astype(jnp.float32)
        mm = (pos[:, None] == cols).astype(jnp.float32)
        acc = jnp.dot(lhs, mm, preferred_element_type=jnp.float32)
        srcc = 16.0 * acc[0:1] + acc[1:2]
        dstc = srcc + (1.0 - acc[2:3]) * float(S)
        src_parts.append(srcc)
        dst_parts.append(dstc)
    src = jnp.concatenate(src_parts, axis=1)
    dst = jnp.concatenate(dst_parts, axis=1)
    src_ref[...] = jnp.round(src).astype(jnp.int32)
    dst_ref[...] = jnp.round(dst).astype(jnp.int32)

    gt = lax.broadcasted_iota(jnp.int32, (1, NB), 1) * T
    ge = (gt[:, :, None] >= cum_incl[None, None, :]).astype(jnp.int32)
    bexp_ref[...] = jnp.minimum(jnp.sum(ge, axis=2), E - 1)


def _router_call(x2, wr):
    return pl.pallas_call(
        _router_kernel,
        out_shape=(
            jax.ShapeDtypeStruct((1, SPAD), jnp.int32),
            jax.ShapeDtypeStruct((1, SPAD), jnp.int32),
            jax.ShapeDtypeStruct((1, NB), jnp.int32),
        ),
    )(x2, wr)


@functools.cache
def _sc_kernels():
    mesh = plsc.VectorSubcoreMesh(core_axis_name="c", subcore_axis_name="s")

    @functools.partial(
        pl.kernel,
        mesh=mesh,
        out_type=jax.ShapeDtypeStruct((SPAD, D), jnp.float32),
        scratch_types=[
            pltpu.VMEM((RPW,), jnp.int32),
            pltpu.VMEM((RPW, D), jnp.float32),
            pltpu.SemaphoreType.DMA,
        ],
    )
    def _sc_gather(src_hbm, x_hbm, out_hbm, idx_v, rows_v, sem):
        wid = lax.axis_index("s") * NC + lax.axis_index("c")
        base = wid * RPW
        pltpu.sync_copy(src_hbm.at[pl.ds(base, RPW)], idx_v)
        pltpu.async_copy(x_hbm.at[idx_v], rows_v, sem).wait()
        pltpu.sync_copy(rows_v, out_hbm.at[pl.ds(base, RPW)])

    @functools.partial(
        pl.kernel,
        mesh=mesh,
        out_type=jax.ShapeDtypeStruct((S + 8, D), jnp.float32),
        scratch_types=[
            pltpu.VMEM((RPW,), jnp.int32),
            pltpu.VMEM((RPW, D), jnp.float32),
            pltpu.SemaphoreType.DMA,
        ],
    )
    def _sc_scatter(ys_hbm, dst_hbm, out_hbm, idx_v, rows_v, sem):
        wid = lax.axis_index("s") * NC + lax.axis_index("c")
        base = wid * RPW
        pltpu.sync_copy(dst_hbm.at[pl.ds(base, RPW)], idx_v)
        pltpu.sync_copy(ys_hbm.at[pl.ds(base, RPW)], rows_v)
        pltpu.async_copy(rows_v, out_hbm.at[idx_v], sem).wait()

    return _sc_gather, _sc_scatter


def _gmm_kernel(bexp_ref, xs_ref, wg_ref, wu_ref, wd_ref, ys_ref):
    xb = xs_ref[...]
    g = jnp.dot(xb, wg_ref[0], preferred_element_type=jnp.float32)
    u = jnp.dot(xb, wu_ref[0], preferred_element_type=jnp.float32)
    h = g * lax.logistic(g) * u
    ys_ref[...] = jnp.dot(h, wd_ref[0], preferred_element_type=jnp.float32)


def _gmm_call(bexp, xs, wg, wu, wd):
    grid_spec = pltpu.PrefetchScalarGridSpec(
        num_scalar_prefetch=1,
        grid=(NB,),
        in_specs=[
            pl.BlockSpec((T, D), lambda g, be: (g, 0)),
            pl.BlockSpec((1, D, F), lambda g, be: (be[g], 0, 0)),
            pl.BlockSpec((1, D, F), lambda g, be: (be[g], 0, 0)),
            pl.BlockSpec((1, F, D), lambda g, be: (be[g], 0, 0)),
        ],
        out_specs=pl.BlockSpec((T, D), lambda g, be: (g, 0)),
    )
    return pl.pallas_call(
        _gmm_kernel,
        grid_spec=grid_spec,
        out_shape=jax.ShapeDtypeStruct((SPAD, D), jnp.float32),
    )(bexp, xs, wg, wu, wd)


_TB = 256

def _shared_kernel(x_ref, yr_ref, wsg_ref, wsu_ref, wsd_ref, y_ref):
    xb = x_ref[...]
    g = jnp.dot(xb, wsg_ref[...], preferred_element_type=jnp.float32)
    u = jnp.dot(xb, wsu_ref[...], preferred_element_type=jnp.float32)
    h = g * lax.logistic(g) * u
    y_ref[...] = (jnp.dot(h, wsd_ref[...], preferred_element_type=jnp.float32)
                  + yr_ref[...])


def _shared_call(x2, yr, wsg, wsu, wsd):
    return pl.pallas_call(
        _shared_kernel,
        grid=(S // _TB,),
        in_specs=[
            pl.BlockSpec((_TB, D), lambda g: (g, 0)),
            pl.BlockSpec((_TB, D), lambda g: (g, 0)),
            pl.BlockSpec((D, F), lambda g: (0, 0)),
            pl.BlockSpec((D, F), lambda g: (0, 0)),
            pl.BlockSpec((F, D), lambda g: (0, 0)),
        ],
        out_specs=pl.BlockSpec((_TB, D), lambda g: (g, 0)),
        out_shape=jax.ShapeDtypeStruct((S, D), jnp.float32),
    )(x2, yr, wsg, wsu, wsd)


def kernel(x, Wr, Wg, Wu, Wd, Wsg, Wsu, Wsd):
    x2 = x.reshape(S, D)
    _sc_gather, _sc_scatter = _sc_kernels()
    src, dst, bexp = _router_call(x2, Wr)
    xs = _sc_gather(src.reshape(SPAD), x2)
    ys = _gmm_call(bexp.reshape(NB), xs, Wg, Wu, Wd)
    ybuf = _sc_scatter(ys, dst.reshape(SPAD))
    y = _shared_call(x2, ybuf[:S], Wsg, Wsu, Wsd)
    return y.reshape(1, S, D)
